# baseline (device time: 175145 ns/iter reference)
import jax
import jax.numpy as jnp
from jax import lax
from jax.experimental import pallas as pl
from jax.experimental.pallas import tpu as pltpu

N_DEV = 4
M = 4096
D = 1024
M_BLK = M // N_DEV


def _allreduce_body(h_ref, out_ref, rs_buf, rs_send_sems, rs_recv_sems,
                    ag_send_sems, ag_recv_sems):
    my = lax.axis_index("i")
    left = (my - 1) % N_DEV
    right = (my + 1) % N_DEV

    barrier_sem = pltpu.get_barrier_semaphore()
    for nbr in (left, right):
        pl.semaphore_signal(
            barrier_sem, inc=1,
            device_id=(nbr,), device_id_type=pl.DeviceIdType.MESH,
        )
    pl.semaphore_wait(barrier_sem, 2)

    for s in range(N_DEV - 1):
        chunk_send = (my - s) % N_DEV
        if s == 0:
            src = h_ref.at[pl.ds(chunk_send * M_BLK, M_BLK), :]
        else:
            src = rs_buf.at[s - 1]
        rdma = pltpu.make_async_remote_copy(
            src_ref=src,
            dst_ref=rs_buf.at[s],
            send_sem=rs_send_sems.at[s],
            recv_sem=rs_recv_sems.at[s],
            device_id=(right,),
            device_id_type=pl.DeviceIdType.MESH,
        )
        rdma.start()
        rdma.wait()

        chunk_recv = (my - s - 1) % N_DEV
        rs_buf[s, :, :] = (
            rs_buf[s, :, :] + h_ref[pl.ds(chunk_recv * M_BLK, M_BLK), :]
        )

    owned = (my + 1) % N_DEV
    out_ref[pl.ds(owned * M_BLK, M_BLK), :] = rs_buf[N_DEV - 2, :, :]

    for t in range(N_DEV - 1):
        chunk_send = (my + 1 - t) % N_DEV
        if t == 0:
            src = rs_buf.at[N_DEV - 2]
        else:
            src = out_ref.at[pl.ds(chunk_send * M_BLK, M_BLK), :]
        rdma = pltpu.make_async_remote_copy(
            src_ref=src,
            dst_ref=out_ref.at[pl.ds(chunk_send * M_BLK, M_BLK), :],
            send_sem=ag_send_sems.at[t],
            recv_sem=ag_recv_sems.at[t],
            device_id=(right,),
            device_id_type=pl.DeviceIdType.MESH,
        )
        rdma.start()
        rdma.wait()


def _allreduce(h_partial):
    return pl.pallas_call(
        _allreduce_body,
        out_shape=jax.ShapeDtypeStruct((M, D), h_partial.dtype),
        in_specs=[pl.BlockSpec(memory_space=pltpu.VMEM)],
        out_specs=pl.BlockSpec(memory_space=pltpu.VMEM),
        scratch_shapes=[
            pltpu.VMEM((N_DEV - 1, M_BLK, D), h_partial.dtype),
            pltpu.SemaphoreType.DMA((N_DEV - 1,)),
            pltpu.SemaphoreType.DMA((N_DEV - 1,)),
            pltpu.SemaphoreType.DMA((N_DEV - 1,)),
            pltpu.SemaphoreType.DMA((N_DEV - 1,)),
        ],
        compiler_params=pltpu.CompilerParams(collective_id=0),
    )(h_partial)


def kernel(x, W1, W2):
    xb = x.astype(jnp.bfloat16)
    W1b = W1.astype(jnp.bfloat16)
    W2b = W2.astype(jnp.bfloat16)

    h_partial = jnp.dot(
        xb, W1b, preferred_element_type=jnp.float32
    ).astype(jnp.bfloat16)

    h = _allreduce(h_partial)

    out = jnp.dot(h, W2b, preferred_element_type=jnp.float32)
    return out


# device time: 107555 ns/iter; 1.6284x vs baseline; 1.6284x over previous
import jax
import jax.numpy as jnp
from jax import lax
from jax.experimental import pallas as pl
from jax.experimental.pallas import tpu as pltpu

N_DEV = 4
M = 4096
D = 1024
M_BLK = M // N_DEV
D_HALF = D // 2


def _allreduce_body(h_ref, out_ref, rs_r, rs_l,
                    sems_sr, sems_rr, sems_sl, sems_rl,
                    ag_sr, ag_rr, ag_sl, ag_rl):
    my = lax.axis_index("i")
    left = (my - 1) % N_DEV
    right = (my + 1) % N_DEV

    barrier_sem = pltpu.get_barrier_semaphore()
    for nbr in (left, right):
        pl.semaphore_signal(
            barrier_sem, inc=1,
            device_id=(nbr,), device_id_type=pl.DeviceIdType.MESH,
        )
    pl.semaphore_wait(barrier_sem, 2)

    for s in range(N_DEV - 1):
        ch_r = (my - s) % N_DEV
        ch_l = (my + s) % N_DEV
        if s == 0:
            src_r = h_ref.at[pl.ds(ch_r * M_BLK, M_BLK), pl.ds(0, D_HALF)]
            src_l = h_ref.at[pl.ds(ch_l * M_BLK, M_BLK), pl.ds(D_HALF, D_HALF)]
        else:
            src_r = rs_r.at[s - 1]
            src_l = rs_l.at[s - 1]
        rdma_r = pltpu.make_async_remote_copy(
            src_ref=src_r, dst_ref=rs_r.at[s],
            send_sem=sems_sr.at[s], recv_sem=sems_rr.at[s],
            device_id=(right,), device_id_type=pl.DeviceIdType.MESH,
        )
        rdma_l = pltpu.make_async_remote_copy(
            src_ref=src_l, dst_ref=rs_l.at[s],
            send_sem=sems_sl.at[s], recv_sem=sems_rl.at[s],
            device_id=(left,), device_id_type=pl.DeviceIdType.MESH,
        )
        rdma_r.start()
        rdma_l.start()
        rdma_r.wait()
        rdma_l.wait()

        rcv_r = (my - s - 1) % N_DEV
        rcv_l = (my + s + 1) % N_DEV
        rs_r[s, :, :] = (
            rs_r[s, :, :] + h_ref[pl.ds(rcv_r * M_BLK, M_BLK), pl.ds(0, D_HALF)]
        )
        rs_l[s, :, :] = (
            rs_l[s, :, :]
            + h_ref[pl.ds(rcv_l * M_BLK, M_BLK), pl.ds(D_HALF, D_HALF)]
        )

    own_r = (my + 1) % N_DEV
    own_l = (my - 1) % N_DEV
    out_ref[pl.ds(own_r * M_BLK, M_BLK), pl.ds(0, D_HALF)] = rs_r[N_DEV - 2]
    out_ref[pl.ds(own_l * M_BLK, M_BLK), pl.ds(D_HALF, D_HALF)] = rs_l[N_DEV - 2]

    for t in range(N_DEV - 1):
        ch_r = (my + 1 - t) % N_DEV
        ch_l = (my - 1 + t) % N_DEV
        if t == 0:
            src_r = rs_r.at[N_DEV - 2]
            src_l = rs_l.at[N_DEV - 2]
        else:
            src_r = out_ref.at[pl.ds(ch_r * M_BLK, M_BLK), pl.ds(0, D_HALF)]
            src_l = out_ref.at[pl.ds(ch_l * M_BLK, M_BLK), pl.ds(D_HALF, D_HALF)]
        rdma_r = pltpu.make_async_remote_copy(
            src_ref=src_r,
            dst_ref=out_ref.at[pl.ds(ch_r * M_BLK, M_BLK), pl.ds(0, D_HALF)],
            send_sem=ag_sr.at[t], recv_sem=ag_rr.at[t],
            device_id=(right,), device_id_type=pl.DeviceIdType.MESH,
        )
        rdma_l = pltpu.make_async_remote_copy(
            src_ref=src_l,
            dst_ref=out_ref.at[pl.ds(ch_l * M_BLK, M_BLK), pl.ds(D_HALF, D_HALF)],
            send_sem=ag_sl.at[t], recv_sem=ag_rl.at[t],
            device_id=(left,), device_id_type=pl.DeviceIdType.MESH,
        )
        rdma_r.start()
        rdma_l.start()
        rdma_r.wait()
        rdma_l.wait()


def _allreduce(h_partial):
    sem3 = pltpu.SemaphoreType.DMA((N_DEV - 1,))
    return pl.pallas_call(
        _allreduce_body,
        out_shape=jax.ShapeDtypeStruct((M, D), h_partial.dtype),
        in_specs=[pl.BlockSpec(memory_space=pltpu.VMEM)],
        out_specs=pl.BlockSpec(memory_space=pltpu.VMEM),
        scratch_shapes=[
            pltpu.VMEM((N_DEV - 1, M_BLK, D_HALF), h_partial.dtype),
            pltpu.VMEM((N_DEV - 1, M_BLK, D_HALF), h_partial.dtype),
            sem3, sem3, sem3, sem3,
            sem3, sem3, sem3, sem3,
        ],
        compiler_params=pltpu.CompilerParams(collective_id=0),
    )(h_partial)


def kernel(x, W1, W2):
    xb = x.astype(jnp.bfloat16)
    W1b = W1.astype(jnp.bfloat16)
    W2b = W2.astype(jnp.bfloat16)

    h_partial = jnp.dot(
        xb, W1b, preferred_element_type=jnp.float32
    ).astype(jnp.bfloat16)

    h = _allreduce(h_partial)

    out = jnp.dot(h, W2b, preferred_element_type=jnp.float32)
    return out
